# Initial kernel scaffold; baseline (speedup 1.0000x reference)
import jax
import jax.numpy as jnp
from jax.experimental import pallas as pl


def kernel(x, edge_index, edge_attr, batch, t_cond, eps0, We0, be0, W1_0, b1_0, W2_0, b2_0, eps1, We1, be1, W1_1, b1_1, W2_1, b2_1, eps2, We2, be2, W1_2, b1_2, W2_2, b2_2, Wh1, bh1, Wh2, bh2):
    raise NotImplementedError("write your pallas kernel here")


# speedup vs baseline: 2.3797x; 2.3797x over previous
import functools

import jax
import jax.numpy as jnp
from jax import lax
from jax.experimental import pallas as pl
from jax.experimental.pallas import tpu as pltpu
from jax.experimental.pallas import tpu_sc as plsc

_N = 10000
_E = 320000
_D = 128
_ED = 16
_H = 128
_G = 64

_LANES = 16
_NC = 2
_NS = 16
_NW = _NC * _NS
_EP = _E // _NW
_CH = 80
_NCHUNK = _EP // _CH
_NP = 10240
_RPT = _NP // _NS
_ZR = 128


def _edge_agg_body(h_hbm, src_hbm, dst_hbm, e_hbm, out_hbm,
                   agg_sh, src_v, dst_v, g_v, e_v, z_v, sem):
    c = lax.axis_index("c")
    s = lax.axis_index("s")
    wid = c * _NS + s

    def zrow(i, carry):
        for k in range(_D // _LANES):
            z_v[i, pl.ds(k * _LANES, _LANES)] = jnp.zeros((_LANES,), jnp.float32)
        return carry

    lax.fori_loop(0, _ZR, zrow, 0)

    def zcopy(j, carry):
        pltpu.sync_copy(z_v, agg_sh.at[pl.ds(s * _RPT + j * _ZR, _ZR)])
        return carry

    lax.fori_loop(0, _RPT // _ZR, zcopy, 0)
    plsc.subcore_barrier()

    ebase = wid * _EP

    def chunk(j, carry):
        base = ebase + j * _CH
        pltpu.sync_copy(src_hbm.at[pl.ds(base, _CH)], src_v)
        pltpu.sync_copy(dst_hbm.at[pl.ds(base, _CH)], dst_v)
        pltpu.sync_copy(e_hbm.at[pl.ds(base, _CH)], e_v)
        pltpu.async_copy(h_hbm.at[src_v], g_v, sem).wait()

        def row(r, rc):
            for k in range(_D // _LANES):
                sl = pl.ds(k * _LANES, _LANES)
                g_v[r, sl] = jnp.maximum(g_v[r, sl] + e_v[r, sl], 0.0)
            return rc

        lax.fori_loop(0, _CH, row, 0)
        pltpu.sync_copy(g_v, agg_sh.at[dst_v], add=True)
        return carry

    lax.fori_loop(0, _NCHUNK, chunk, 0)

    plsc.subcore_barrier()
    pltpu.sync_copy(agg_sh.at[pl.ds(s * _RPT, _RPT)],
                    out_hbm.at[c, pl.ds(s * _RPT, _RPT)])


def _edge_agg(h, src, dst, e):
    kfn = functools.partial(
        pl.kernel,
        mesh=plsc.VectorSubcoreMesh(core_axis_name="c", subcore_axis_name="s"),
        out_type=jax.ShapeDtypeStruct((_NC, _NP, _D), jnp.float32),
        scratch_types=[
            pltpu.VMEM_SHARED((_NP, _D), jnp.float32),
            pltpu.VMEM((_CH,), jnp.int32),
            pltpu.VMEM((_CH,), jnp.int32),
            pltpu.VMEM((_CH, _D), jnp.float32),
            pltpu.VMEM((_CH, _D), jnp.float32),
            pltpu.VMEM((_ZR, _D), jnp.float32),
            pltpu.SemaphoreType.DMA,
        ],
    )(_edge_agg_body)
    return kfn(h, src, dst, e)


def _edge_proj_body(ea_ref, w_ref, b_ref, o_ref):
    o_ref[...] = (
        jnp.dot(ea_ref[...], w_ref[...], preferred_element_type=jnp.float32,
                precision=lax.Precision.HIGHEST)
        + b_ref[...]
    )


def _edge_proj(ea_packed, w_bd, bias_t):
    rows = ea_packed.shape[0]
    blk = 2000
    grid = rows // blk
    return pl.pallas_call(
        _edge_proj_body,
        grid=(grid,),
        in_specs=[
            pl.BlockSpec((blk, 8 * _ED), lambda i: (i, 0)),
            pl.BlockSpec((8 * _ED, 8 * _H), lambda i: (0, 0)),
            pl.BlockSpec((1, 8 * _H), lambda i: (0, 0)),
        ],
        out_specs=pl.BlockSpec((blk, 8 * _H), lambda i: (i, 0)),
        out_shape=jax.ShapeDtypeStruct((rows, 8 * _H), jnp.float32),
    )(ea_packed, w_bd, bias_t)


def _node_mlp_body(h_ref, p_ref, sc_ref, w1_ref, b1_ref, w2_ref, b2_ref, o_ref):
    hcur = h_ref[...] * sc_ref[0, 0] + p_ref[0] + p_ref[1]
    t = jnp.maximum(
        jnp.dot(hcur, w1_ref[...], preferred_element_type=jnp.float32,
                precision=lax.Precision.HIGHEST) + b1_ref[...], 0.0)
    o_ref[...] = jnp.maximum(
        jnp.dot(t, w2_ref[...], preferred_element_type=jnp.float32,
                precision=lax.Precision.HIGHEST) + b2_ref[...], 0.0)


def _node_mlp(h, partial, scale, w1, b1, w2, b2):
    blk = 2000
    grid = _N // blk
    return pl.pallas_call(
        _node_mlp_body,
        grid=(grid,),
        in_specs=[
            pl.BlockSpec((blk, _D), lambda i: (i, 0)),
            pl.BlockSpec((_NC, blk, _D), lambda i: (0, i, 0)),
            pl.BlockSpec((1, 1), lambda i: (0, 0)),
            pl.BlockSpec((_D, _H), lambda i: (0, 0)),
            pl.BlockSpec((1, _H), lambda i: (0, 0)),
            pl.BlockSpec((_H, _H), lambda i: (0, 0)),
            pl.BlockSpec((1, _H), lambda i: (0, 0)),
        ],
        out_specs=pl.BlockSpec((blk, _H), lambda i: (i, 0)),
        out_shape=jax.ShapeDtypeStruct((_N, _H), jnp.float32),
    )(h, partial, scale, w1, b1, w2, b2)


def _pool_head_body(h_ref, b_ref, t_ref, w1_ref, wrow_ref, b1_ref,
                    w2_ref, b2_ref, o_ref, pooled_acc, cnt_acc):
    i = pl.program_id(0)
    nsteps = pl.num_programs(0)

    @pl.when(i == 0)
    def _init():
        pooled_acc[...] = jnp.zeros_like(pooled_acc)
        cnt_acc[...] = jnp.zeros_like(cnt_acc)

    b = b_ref[0, 0, :]
    oh = jnp.equal(
        b[:, None],
        lax.broadcasted_iota(jnp.int32, (b.shape[0], _G), 1),
    ).astype(jnp.float32)
    pooled_acc[...] += lax.dot_general(
        oh, h_ref[...], (((0,), (0,)), ((), ())),
        preferred_element_type=jnp.float32, precision=lax.Precision.HIGHEST)
    cnt_acc[...] += lax.dot_general(
        oh, jnp.ones_like(h_ref), (((0,), (0,)), ((), ())),
        preferred_element_type=jnp.float32, precision=lax.Precision.HIGHEST)

    @pl.when(i == nsteps - 1)
    def _final():
        mean = pooled_acc[...] / jnp.maximum(cnt_acc[...], 1.0)
        r1 = jnp.maximum(
            jnp.dot(mean, w1_ref[...], preferred_element_type=jnp.float32,
                    precision=lax.Precision.HIGHEST)
            + t_ref[...] * wrow_ref[...] + b1_ref[...], 0.0)
        o_ref[...] = (
            jnp.dot(r1, w2_ref[...], preferred_element_type=jnp.float32,
                    precision=lax.Precision.HIGHEST) + b2_ref[...])


def _pool_head(h, batch_p, t_col, w1a, wrow, b1, w2pad, b2pad):
    blk = 2000
    grid = _N // blk
    return pl.pallas_call(
        _pool_head_body,
        grid=(grid,),
        in_specs=[
            pl.BlockSpec((blk, _D), lambda i: (i, 0)),
            pl.BlockSpec((1, 1, blk), lambda i: (i, 0, 0)),
            pl.BlockSpec((_G, _H), lambda i: (0, 0)),
            pl.BlockSpec((_H, _H), lambda i: (0, 0)),
            pl.BlockSpec((1, _H), lambda i: (0, 0)),
            pl.BlockSpec((1, _H), lambda i: (0, 0)),
            pl.BlockSpec((_H, _H), lambda i: (0, 0)),
            pl.BlockSpec((1, _H), lambda i: (0, 0)),
        ],
        out_specs=pl.BlockSpec((_G, _H), lambda i: (0, 0)),
        out_shape=jax.ShapeDtypeStruct((_G, _H), jnp.float32),
        scratch_shapes=[
            pltpu.VMEM((_G, _H), jnp.float32),
            pltpu.VMEM((_G, _H), jnp.float32),
        ],
    )(h, batch_p, t_col, w1a, wrow, b1, w2pad, b2pad)


def kernel(x, edge_index, edge_attr, batch, t_cond,
           eps0, We0, be0, W1_0, b1_0, W2_0, b2_0,
           eps1, We1, be1, W1_1, b1_1, W2_1, b2_1,
           eps2, We2, be2, W1_2, b1_2, W2_2, b2_2,
           Wh1, bh1, Wh2, bh2):
    src = edge_index[0]
    dst = edge_index[1]
    ea_packed = edge_attr.reshape(_E // 8, 8 * _ED)
    eye8 = jnp.eye(8, dtype=jnp.float32)

    h = x
    layers = [
        (eps0, We0, be0, W1_0, b1_0, W2_0, b2_0),
        (eps1, We1, be1, W1_1, b1_1, W2_1, b2_1),
        (eps2, We2, be2, W1_2, b1_2, W2_2, b2_2),
    ]
    for (eps, We, be, W1, b1, W2, b2) in layers:
        w_bd = jnp.kron(eye8, We)
        bias_t = jnp.tile(be, 8).reshape(1, 8 * _H)
        e = _edge_proj(ea_packed, w_bd, bias_t).reshape(_E, _H)
        partial = _edge_agg(h, src, dst, e)
        h = _node_mlp(h, partial, (1.0 + eps).reshape(1, 1),
                      W1, b1.reshape(1, _H), W2, b2.reshape(1, _H))

    batch_p = batch.reshape(_N // 2000, 1, 2000)
    t_col = jnp.broadcast_to(t_cond[:, None], (_G, _H))
    w2pad = jnp.pad(Wh2, ((0, 0), (0, _H - 1)))
    b2pad = jnp.pad(bh2.reshape(1, 1), ((0, 0), (0, _H - 1)))
    out128 = _pool_head(h, batch_p, t_col, Wh1[:_H], Wh1[_H:_H + 1],
                        bh1.reshape(1, _H), w2pad, b2pad)
    return out128[:, :1]


# speedup vs baseline: 4.3729x; 1.8376x over previous
import functools

import jax
import jax.numpy as jnp
import numpy as np
from jax import lax
from jax.experimental import pallas as pl
from jax.experimental.pallas import tpu as pltpu
from jax.experimental.pallas import tpu_sc as plsc

_N = 10000
_E = 320000
_D = 128
_ED = 16
_H = 128
_G = 64

_LANES = 16
_NC = 2
_NS = 16
_NW = _NC * _NS
_EP = _E // _NW
_CH = 80
_NCHUNK = _EP // _CH
_NP = 10240
_RPT = _NP // _NS


def _edge_agg_body(h_hbm, src_hbm, dst_hbm, e_hbm, out_hbm,
                   agg_sh, s0, s1, d0, d1, g0, g1, e0, e1,
                   ss0, ss1, ds0, ds1, gs0, gs1, es0, es1):
    c = lax.axis_index("c")
    s = lax.axis_index("s")
    wid = c * _NS + s

    sidx = (s0, s1)
    dbuf = (d0, d1)
    gbuf = (g0, g1)
    ebuf = (e0, e1)
    ssem = (ss0, ss1)
    dsem = (ds0, ds1)
    gsem = (gs0, gs1)
    esem = (es0, es1)
    ebase = wid * _EP
    cbase = wid * _NCHUNK

    def sidx_copy(j, b):
        return pltpu.make_async_copy(src_hbm.at[pl.ds(ebase + j * _CH, _CH)],
                                     sidx[b], ssem[b])

    def didx_copy(j, b):
        return pltpu.make_async_copy(dst_hbm.at[pl.ds(ebase + j * _CH, _CH)],
                                     dbuf[b], dsem[b])

    def gather_copy(b):
        return pltpu.make_async_copy(h_hbm.at[sidx[b]], gbuf[b], gsem[b])

    def e_copy(j, b):
        return pltpu.make_async_copy(e_hbm.at[pl.ds(ebase + j * _CH, _CH)],
                                     ebuf[b], esem[b])

    def zrow(i, carry):
        for k in range(_D // _LANES):
            g0[i, pl.ds(k * _LANES, _LANES)] = jnp.zeros((_LANES,), jnp.float32)
        return carry

    lax.fori_loop(0, _CH, zrow, 0)

    def zcopy(j, carry):
        pltpu.sync_copy(g0, agg_sh.at[pl.ds(s * _RPT + j * _CH, _CH)])
        return carry

    lax.fori_loop(0, _RPT // _CH, zcopy, 0)
    plsc.subcore_barrier()

    sidx_copy(0, 0).start()
    sidx_copy(1, 1).start()
    didx_copy(0, 0).start()
    didx_copy(1, 1).start()
    sidx_copy(0, 0).wait()
    gather_copy(0).start()
    e_copy(0, 0).start()

    def process(j, b):
        nb = 1 - b

        @pl.when(j + 1 < _NCHUNK)
        def _issue_next_gather():
            sidx_copy(j + 1, nb).wait()
            gather_copy(nb).start()
            e_copy(j + 1, nb).start()

        gather_copy(b).wait()
        e_copy(j, b).wait()

        @pl.when(j + 2 < _NCHUNK)
        def _issue_next_sidx():
            sidx_copy(j + 2, b).start()

        def row(r, rc):
            for k in range(_D // _LANES):
                sl = pl.ds(k * _LANES, _LANES)
                gbuf[b][r, sl] = jnp.maximum(gbuf[b][r, sl] + ebuf[b][r, sl],
                                             0.0)
            return rc

        lax.fori_loop(0, _CH, row, 0)
        didx_copy(j, b).wait()
        pltpu.sync_copy(gbuf[b], agg_sh.at[dbuf[b]], add=True)

        @pl.when(j + 2 < _NCHUNK)
        def _issue_next_didx():
            didx_copy(j + 2, b).start()

    def pair(j2, carry):
        process(2 * j2, 0)
        process(2 * j2 + 1, 1)
        return carry

    lax.fori_loop(0, _NCHUNK // 2, pair, 0)
    process(_NCHUNK - 1, 0)

    plsc.subcore_barrier()
    pltpu.sync_copy(agg_sh.at[pl.ds(s * _RPT, _RPT)],
                    out_hbm.at[c, pl.ds(s * _RPT, _RPT)])


def _edge_agg(h, src3, dst3, e):
    kfn = functools.partial(
        pl.kernel,
        mesh=plsc.VectorSubcoreMesh(core_axis_name="c", subcore_axis_name="s"),
        out_type=jax.ShapeDtypeStruct((_NC, _NP, _D), jnp.float32),
        scratch_types=[
            pltpu.VMEM_SHARED((_NP, _D), jnp.float32),
            pltpu.VMEM((_CH,), jnp.int32),
            pltpu.VMEM((_CH,), jnp.int32),
            pltpu.VMEM((_CH,), jnp.int32),
            pltpu.VMEM((_CH,), jnp.int32),
            pltpu.VMEM((_CH, _D), jnp.float32),
            pltpu.VMEM((_CH, _D), jnp.float32),
            pltpu.VMEM((_CH, _D), jnp.float32),
            pltpu.VMEM((_CH, _D), jnp.float32),
            pltpu.SemaphoreType.DMA,
            pltpu.SemaphoreType.DMA,
            pltpu.SemaphoreType.DMA,
            pltpu.SemaphoreType.DMA,
            pltpu.SemaphoreType.DMA,
            pltpu.SemaphoreType.DMA,
            pltpu.SemaphoreType.DMA,
            pltpu.SemaphoreType.DMA,
        ],
    )(_edge_agg_body)
    return kfn(h, src3, dst3, e)


def _edge_proj_body(ea_ref, w_ref, b_ref, o_ref):
    o_ref[...] = (
        jnp.dot(ea_ref[...], w_ref[...], preferred_element_type=jnp.float32,
                precision=lax.Precision.DEFAULT)
        + b_ref[...]
    )


def _edge_proj(ea_packed, w_bd, bias_t):
    rows = ea_packed.shape[0]
    blk = 2000
    grid = rows // blk
    return pl.pallas_call(
        _edge_proj_body,
        grid=(grid,),
        in_specs=[
            pl.BlockSpec((blk, 8 * _ED), lambda i: (i, 0)),
            pl.BlockSpec((8 * _ED, 8 * _H), lambda i: (0, 0)),
            pl.BlockSpec((1, 8 * _H), lambda i: (0, 0)),
        ],
        out_specs=pl.BlockSpec((blk, 8 * _H), lambda i: (i, 0)),
        out_shape=jax.ShapeDtypeStruct((rows, 8 * _H), jnp.float32),
    )(ea_packed, w_bd, bias_t)


def _node_mlp_body(h_ref, p_ref, sc_ref, w1_ref, b1_ref, w2_ref, b2_ref, o_ref):
    hcur = h_ref[...] * sc_ref[0, 0] + p_ref[0] + p_ref[1]
    t = jnp.maximum(
        jnp.dot(hcur, w1_ref[...], preferred_element_type=jnp.float32,
                precision=lax.Precision.DEFAULT) + b1_ref[...], 0.0)
    o_ref[...] = jnp.maximum(
        jnp.dot(t, w2_ref[...], preferred_element_type=jnp.float32,
                precision=lax.Precision.DEFAULT) + b2_ref[...], 0.0)


def _node_mlp(h, partial, scale, w1, b1, w2, b2):
    blk = 2000
    grid = _N // blk
    return pl.pallas_call(
        _node_mlp_body,
        grid=(grid,),
        in_specs=[
            pl.BlockSpec((blk, _D), lambda i: (i, 0)),
            pl.BlockSpec((_NC, blk, _D), lambda i: (0, i, 0)),
            pl.BlockSpec((1, 1), lambda i: (0, 0)),
            pl.BlockSpec((_D, _H), lambda i: (0, 0)),
            pl.BlockSpec((1, _H), lambda i: (0, 0)),
            pl.BlockSpec((_H, _H), lambda i: (0, 0)),
            pl.BlockSpec((1, _H), lambda i: (0, 0)),
        ],
        out_specs=pl.BlockSpec((blk, _H), lambda i: (i, 0)),
        out_shape=jax.ShapeDtypeStruct((_N, _H), jnp.float32),
    )(h, partial, scale, w1, b1, w2, b2)


def _pool_head_body(h_ref, b_ref, t_ref, w1_ref, wrow_ref, b1_ref,
                    w2_ref, b2_ref, o_ref, pooled_acc, cnt_acc):
    i = pl.program_id(0)
    nsteps = pl.num_programs(0)

    @pl.when(i == 0)
    def _init():
        pooled_acc[...] = jnp.zeros_like(pooled_acc)
        cnt_acc[...] = jnp.zeros_like(cnt_acc)

    b = b_ref[0, 0, :]
    oh = jnp.equal(
        b[:, None],
        lax.broadcasted_iota(jnp.int32, (b.shape[0], _G), 1),
    ).astype(jnp.float32)
    pooled_acc[...] += lax.dot_general(
        oh, h_ref[...], (((0,), (0,)), ((), ())),
        preferred_element_type=jnp.float32, precision=lax.Precision.HIGHEST)
    cnt_acc[...] += lax.dot_general(
        oh, jnp.ones_like(h_ref), (((0,), (0,)), ((), ())),
        preferred_element_type=jnp.float32, precision=lax.Precision.HIGHEST)

    @pl.when(i == nsteps - 1)
    def _final():
        mean = pooled_acc[...] / jnp.maximum(cnt_acc[...], 1.0)
        r1 = jnp.maximum(
            jnp.dot(mean, w1_ref[...], preferred_element_type=jnp.float32,
                    precision=lax.Precision.DEFAULT)
            + t_ref[...] * wrow_ref[...] + b1_ref[...], 0.0)
        o_ref[...] = (
            jnp.dot(r1, w2_ref[...], preferred_element_type=jnp.float32,
                    precision=lax.Precision.DEFAULT) + b2_ref[...])


def _pool_head(h, batch_p, t_col, w1a, wrow, b1, w2pad, b2pad):
    blk = 2000
    grid = _N // blk
    return pl.pallas_call(
        _pool_head_body,
        grid=(grid,),
        in_specs=[
            pl.BlockSpec((blk, _D), lambda i: (i, 0)),
            pl.BlockSpec((1, 1, blk), lambda i: (i, 0, 0)),
            pl.BlockSpec((_G, _H), lambda i: (0, 0)),
            pl.BlockSpec((_H, _H), lambda i: (0, 0)),
            pl.BlockSpec((1, _H), lambda i: (0, 0)),
            pl.BlockSpec((1, _H), lambda i: (0, 0)),
            pl.BlockSpec((_H, _H), lambda i: (0, 0)),
            pl.BlockSpec((1, _H), lambda i: (0, 0)),
        ],
        out_specs=pl.BlockSpec((_G, _H), lambda i: (0, 0)),
        out_shape=jax.ShapeDtypeStruct((_G, _H), jnp.float32),
        scratch_shapes=[
            pltpu.VMEM((_G, _H), jnp.float32),
            pltpu.VMEM((_G, _H), jnp.float32),
        ],
    )(h, batch_p, t_col, w1a, wrow, b1, w2pad, b2pad)


def kernel(x, edge_index, edge_attr, batch, t_cond,
           eps0, We0, be0, W1_0, b1_0, W2_0, b2_0,
           eps1, We1, be1, W1_1, b1_1, W2_1, b2_1,
           eps2, We2, be2, W1_2, b1_2, W2_2, b2_2,
           Wh1, bh1, Wh2, bh2):
    src3 = edge_index[0]
    dst3 = edge_index[1]
    ea_packed = edge_attr.reshape(_E // 8, 8 * _ED)
    eye8 = jnp.eye(8, dtype=jnp.float32)

    h = x
    layers = [
        (eps0, We0, be0, W1_0, b1_0, W2_0, b2_0),
        (eps1, We1, be1, W1_1, b1_1, W2_1, b2_1),
        (eps2, We2, be2, W1_2, b1_2, W2_2, b2_2),
    ]
    for (eps, We, be, W1, b1, W2, b2) in layers:
        w_bd = jnp.kron(eye8, We)
        bias_t = jnp.tile(be, 8).reshape(1, 8 * _H)
        e = _edge_proj(ea_packed, w_bd, bias_t).reshape(_E, _H)
        partial = _edge_agg(h, src3, dst3, e)
        h = _node_mlp(h, partial, (1.0 + eps).reshape(1, 1),
                      W1, b1.reshape(1, _H), W2, b2.reshape(1, _H))

    batch_p = batch.reshape(_N // 2000, 1, 2000)
    t_col = jnp.broadcast_to(t_cond[:, None], (_G, _H))
    w2pad = jnp.pad(Wh2, ((0, 0), (0, _H - 1)))
    b2pad = jnp.pad(bh2.reshape(1, 1), ((0, 0), (0, _H - 1)))
    out128 = _pool_head(h, batch_p, t_col, Wh1[:_H], Wh1[_H:_H + 1],
                        bh1.reshape(1, _H), w2pad, b2pad)
    return out128[:, :1]


# speedup vs baseline: 4.3730x; 1.0000x over previous
import functools

import jax
import jax.numpy as jnp
import numpy as np
from jax import lax
from jax.experimental import pallas as pl
from jax.experimental.pallas import tpu as pltpu
from jax.experimental.pallas import tpu_sc as plsc

_N = 10000
_E = 320000
_D = 128
_ED = 16
_H = 128
_G = 64

_LANES = 16
_NC = 2
_NS = 16
_NW = _NC * _NS
_EP = _E // _NW
_CH = 80
_NCHUNK = _EP // _CH
_NP = 10240
_RPT = _NP // _NS


def _edge_agg_body(h_hbm, src_hbm, dst_hbm, e_hbm, out_hbm,
                   agg_sh, s0, s1, d0, d1, g0, g1, e0, e1,
                   ss0, ss1, ds0, ds1, gs0, gs1, es0, es1, cs0, cs1):
    c = lax.axis_index("c")
    s = lax.axis_index("s")
    wid = c * _NS + s

    sidx = (s0, s1)
    dbuf = (d0, d1)
    gbuf = (g0, g1)
    ebuf = (e0, e1)
    ssem = (ss0, ss1)
    dsem = (ds0, ds1)
    gsem = (gs0, gs1)
    esem = (es0, es1)
    csem = (cs0, cs1)
    ebase = wid * _EP
    cbase = wid * _NCHUNK

    def sidx_copy(j, b):
        return pltpu.make_async_copy(src_hbm.at[pl.ds(ebase + j * _CH, _CH)],
                                     sidx[b], ssem[b])

    def didx_copy(j, b):
        return pltpu.make_async_copy(dst_hbm.at[pl.ds(ebase + j * _CH, _CH)],
                                     dbuf[b], dsem[b])

    def gather_copy(b):
        return pltpu.make_async_copy(h_hbm.at[sidx[b]], gbuf[b], gsem[b])

    def e_copy(j, b):
        return pltpu.make_async_copy(e_hbm.at[pl.ds(ebase + j * _CH, _CH)],
                                     ebuf[b], esem[b])

    def zrow(i, carry):
        for k in range(_D // _LANES):
            g0[i, pl.ds(k * _LANES, _LANES)] = jnp.zeros((_LANES,), jnp.float32)
        return carry

    lax.fori_loop(0, _CH, zrow, 0)

    def zcopy(j, carry):
        pltpu.sync_copy(g0, agg_sh.at[pl.ds(s * _RPT + j * _CH, _CH)])
        return carry

    lax.fori_loop(0, _RPT // _CH, zcopy, 0)
    plsc.subcore_barrier()

    def scat_copy(b):
        return pltpu.make_async_copy(gbuf[b], agg_sh.at[dbuf[b]], csem[b])

    sidx_copy(0, 0).start()
    sidx_copy(1, 1).start()
    didx_copy(0, 0).start()
    sidx_copy(0, 0).wait()
    gather_copy(0).start()
    e_copy(0, 0).start()

    def process(j, b):
        nb = 1 - b

        @pl.when(j + 1 < _NCHUNK)
        def _issue_next_gather():
            @pl.when(j >= 1)
            def _drain_prev_scatter():
                scat_copy(nb).wait()

            sidx_copy(j + 1, nb).wait()
            gather_copy(nb).start()
            e_copy(j + 1, nb).start()
            didx_copy(j + 1, nb).start()

        gather_copy(b).wait()
        e_copy(j, b).wait()

        @pl.when(j + 2 < _NCHUNK)
        def _issue_next_sidx():
            sidx_copy(j + 2, b).start()

        def row(r, rc):
            for k in range(_D // _LANES):
                sl = pl.ds(k * _LANES, _LANES)
                gbuf[b][r, sl] = jnp.maximum(gbuf[b][r, sl] + ebuf[b][r, sl],
                                             0.0)
            return rc

        lax.fori_loop(0, _CH, row, 0)
        didx_copy(j, b).wait()
        pltpu.async_copy(gbuf[b], agg_sh.at[dbuf[b]], csem[b], add=True)

    def pair(j2, carry):
        process(2 * j2, 0)
        process(2 * j2 + 1, 1)
        return carry

    lax.fori_loop(0, _NCHUNK // 2, pair, 0)
    process(_NCHUNK - 1, 0)

    scat_copy(1).wait()
    scat_copy(0).wait()
    plsc.subcore_barrier()
    pltpu.sync_copy(agg_sh.at[pl.ds(s * _RPT, _RPT)],
                    out_hbm.at[c, pl.ds(s * _RPT, _RPT)])


def _edge_agg(h, src3, dst3, e):
    kfn = functools.partial(
        pl.kernel,
        mesh=plsc.VectorSubcoreMesh(core_axis_name="c", subcore_axis_name="s"),
        out_type=jax.ShapeDtypeStruct((_NC, _NP, _D), jnp.float32),
        scratch_types=[
            pltpu.VMEM_SHARED((_NP, _D), jnp.float32),
            pltpu.VMEM((_CH,), jnp.int32),
            pltpu.VMEM((_CH,), jnp.int32),
            pltpu.VMEM((_CH,), jnp.int32),
            pltpu.VMEM((_CH,), jnp.int32),
            pltpu.VMEM((_CH, _D), jnp.float32),
            pltpu.VMEM((_CH, _D), jnp.float32),
            pltpu.VMEM((_CH, _D), jnp.float32),
            pltpu.VMEM((_CH, _D), jnp.float32),
            pltpu.SemaphoreType.DMA,
            pltpu.SemaphoreType.DMA,
            pltpu.SemaphoreType.DMA,
            pltpu.SemaphoreType.DMA,
            pltpu.SemaphoreType.DMA,
            pltpu.SemaphoreType.DMA,
            pltpu.SemaphoreType.DMA,
            pltpu.SemaphoreType.DMA,
            pltpu.SemaphoreType.DMA,
            pltpu.SemaphoreType.DMA,
        ],
    )(_edge_agg_body)
    return kfn(h, src3, dst3, e)


def _edge_proj_body(ea_ref, w_ref, b_ref, o_ref):
    o_ref[...] = (
        jnp.dot(ea_ref[...], w_ref[...], preferred_element_type=jnp.float32,
                precision=lax.Precision.DEFAULT)
        + b_ref[...]
    )


def _edge_proj(ea_packed, w_bd, bias_t):
    rows = ea_packed.shape[0]
    blk = 2000
    grid = rows // blk
    return pl.pallas_call(
        _edge_proj_body,
        grid=(grid,),
        in_specs=[
            pl.BlockSpec((blk, 8 * _ED), lambda i: (i, 0)),
            pl.BlockSpec((8 * _ED, 8 * _H), lambda i: (0, 0)),
            pl.BlockSpec((1, 8 * _H), lambda i: (0, 0)),
        ],
        out_specs=pl.BlockSpec((blk, 8 * _H), lambda i: (i, 0)),
        out_shape=jax.ShapeDtypeStruct((rows, 8 * _H), jnp.float32),
    )(ea_packed, w_bd, bias_t)


def _node_mlp_body(h_ref, p_ref, sc_ref, w1_ref, b1_ref, w2_ref, b2_ref, o_ref):
    hcur = h_ref[...] * sc_ref[0, 0] + p_ref[0] + p_ref[1]
    t = jnp.maximum(
        jnp.dot(hcur, w1_ref[...], preferred_element_type=jnp.float32,
                precision=lax.Precision.DEFAULT) + b1_ref[...], 0.0)
    o_ref[...] = jnp.maximum(
        jnp.dot(t, w2_ref[...], preferred_element_type=jnp.float32,
                precision=lax.Precision.DEFAULT) + b2_ref[...], 0.0)


def _node_mlp(h, partial, scale, w1, b1, w2, b2):
    blk = 2000
    grid = _N // blk
    return pl.pallas_call(
        _node_mlp_body,
        grid=(grid,),
        in_specs=[
            pl.BlockSpec((blk, _D), lambda i: (i, 0)),
            pl.BlockSpec((_NC, blk, _D), lambda i: (0, i, 0)),
            pl.BlockSpec((1, 1), lambda i: (0, 0)),
            pl.BlockSpec((_D, _H), lambda i: (0, 0)),
            pl.BlockSpec((1, _H), lambda i: (0, 0)),
            pl.BlockSpec((_H, _H), lambda i: (0, 0)),
            pl.BlockSpec((1, _H), lambda i: (0, 0)),
        ],
        out_specs=pl.BlockSpec((blk, _H), lambda i: (i, 0)),
        out_shape=jax.ShapeDtypeStruct((_N, _H), jnp.float32),
    )(h, partial, scale, w1, b1, w2, b2)


def _pool_head_body(h_ref, b_ref, t_ref, w1_ref, wrow_ref, b1_ref,
                    w2_ref, b2_ref, o_ref, pooled_acc, cnt_acc):
    i = pl.program_id(0)
    nsteps = pl.num_programs(0)

    @pl.when(i == 0)
    def _init():
        pooled_acc[...] = jnp.zeros_like(pooled_acc)
        cnt_acc[...] = jnp.zeros_like(cnt_acc)

    b = b_ref[0, 0, :]
    oh = jnp.equal(
        b[:, None],
        lax.broadcasted_iota(jnp.int32, (b.shape[0], _G), 1),
    ).astype(jnp.float32)
    pooled_acc[...] += lax.dot_general(
        oh, h_ref[...], (((0,), (0,)), ((), ())),
        preferred_element_type=jnp.float32, precision=lax.Precision.HIGHEST)
    cnt_acc[...] += lax.dot_general(
        oh, jnp.ones_like(h_ref), (((0,), (0,)), ((), ())),
        preferred_element_type=jnp.float32, precision=lax.Precision.HIGHEST)

    @pl.when(i == nsteps - 1)
    def _final():
        mean = pooled_acc[...] / jnp.maximum(cnt_acc[...], 1.0)
        r1 = jnp.maximum(
            jnp.dot(mean, w1_ref[...], preferred_element_type=jnp.float32,
                    precision=lax.Precision.DEFAULT)
            + t_ref[...] * wrow_ref[...] + b1_ref[...], 0.0)
        o_ref[...] = (
            jnp.dot(r1, w2_ref[...], preferred_element_type=jnp.float32,
                    precision=lax.Precision.DEFAULT) + b2_ref[...])


def _pool_head(h, batch_p, t_col, w1a, wrow, b1, w2pad, b2pad):
    blk = 2000
    grid = _N // blk
    return pl.pallas_call(
        _pool_head_body,
        grid=(grid,),
        in_specs=[
            pl.BlockSpec((blk, _D), lambda i: (i, 0)),
            pl.BlockSpec((1, 1, blk), lambda i: (i, 0, 0)),
            pl.BlockSpec((_G, _H), lambda i: (0, 0)),
            pl.BlockSpec((_H, _H), lambda i: (0, 0)),
            pl.BlockSpec((1, _H), lambda i: (0, 0)),
            pl.BlockSpec((1, _H), lambda i: (0, 0)),
            pl.BlockSpec((_H, _H), lambda i: (0, 0)),
            pl.BlockSpec((1, _H), lambda i: (0, 0)),
        ],
        out_specs=pl.BlockSpec((_G, _H), lambda i: (0, 0)),
        out_shape=jax.ShapeDtypeStruct((_G, _H), jnp.float32),
        scratch_shapes=[
            pltpu.VMEM((_G, _H), jnp.float32),
            pltpu.VMEM((_G, _H), jnp.float32),
        ],
    )(h, batch_p, t_col, w1a, wrow, b1, w2pad, b2pad)


def kernel(x, edge_index, edge_attr, batch, t_cond,
           eps0, We0, be0, W1_0, b1_0, W2_0, b2_0,
           eps1, We1, be1, W1_1, b1_1, W2_1, b2_1,
           eps2, We2, be2, W1_2, b1_2, W2_2, b2_2,
           Wh1, bh1, Wh2, bh2):
    src3 = edge_index[0]
    dst3 = edge_index[1]
    ea_packed = edge_attr.reshape(_E // 8, 8 * _ED)
    eye8 = jnp.eye(8, dtype=jnp.float32)

    h = x
    layers = [
        (eps0, We0, be0, W1_0, b1_0, W2_0, b2_0),
        (eps1, We1, be1, W1_1, b1_1, W2_1, b2_1),
        (eps2, We2, be2, W1_2, b1_2, W2_2, b2_2),
    ]
    es = []
    for (eps, We, be, W1, b1, W2, b2) in layers:
        w_bd = jnp.kron(eye8, We)
        bias_t = jnp.tile(be, 8).reshape(1, 8 * _H)
        es.append(_edge_proj(ea_packed, w_bd, bias_t).reshape(_E, _H))

    for li, (eps, We, be, W1, b1, W2, b2) in enumerate(layers):
        e = es[li]
        partial = _edge_agg(h, src3, dst3, e)
        h = _node_mlp(h, partial, (1.0 + eps).reshape(1, 1),
                      W1, b1.reshape(1, _H), W2, b2.reshape(1, _H))

    batch_p = batch.reshape(_N // 2000, 1, 2000)
    t_col = jnp.broadcast_to(t_cond[:, None], (_G, _H))
    w2pad = jnp.pad(Wh2, ((0, 0), (0, _H - 1)))
    b2pad = jnp.pad(bh2.reshape(1, 1), ((0, 0), (0, _H - 1)))
    out128 = _pool_head(h, batch_p, t_col, Wh1[:_H], Wh1[_H:_H + 1],
                        bh1.reshape(1, _H), w2pad, b2pad)
    return out128[:, :1]
